# Initial kernel scaffold; baseline (speedup 1.0000x reference)
import jax
import jax.numpy as jnp
from jax.experimental import pallas as pl


def kernel(x, edge_index, W1, b1, W2, b2):
    raise NotImplementedError("write your pallas kernel here")


# speedup vs baseline: 14.8089x; 14.8089x over previous
import functools

import jax
import jax.numpy as jnp
from jax import lax
from jax.experimental import pallas as pl
from jax.experimental.pallas import tpu as pltpu
from jax.experimental.pallas import tpu_sc as plsc

N_NODES = 10000
N_EDGES = 320000
D = 128
DH = D // 2

NC = 2
NS = 16
NW = NC * NS
EPW = N_EDGES // NW
EPT = N_EDGES // NS
K = 80
NB = EPT // K
NPAD = 10240
ROWS_PER_TILE = NPAD // NS
ZCH = 128

_sc_mesh = plsc.VectorSubcoreMesh(core_axis_name="c", subcore_axis_name="s")


@functools.partial(
    pl.kernel,
    out_type=jax.ShapeDtypeStruct((NW, N_NODES), jnp.float32),
    mesh=_sc_mesh,
    scratch_types=[
        pltpu.VMEM((EPW,), jnp.int32),
        pltpu.VMEM((N_NODES,), jnp.float32),
    ],
    compiler_params=pltpu.CompilerParams(needs_layout_passes=False),
)
def _deg_sc(dst_hbm, hist_hbm, dstbuf, hist):
    wid = lax.axis_index("c") * NS + lax.axis_index("s")
    pltpu.sync_copy(dst_hbm.at[wid], dstbuf)
    zeros16 = jnp.zeros((16,), jnp.float32)
    ones16 = jnp.ones((16,), jnp.float32)

    @pl.loop(0, N_NODES // 16)
    def _zero(j):
        hist[pl.ds(j * 16, 16)] = zeros16

    @pl.loop(0, EPW // 16)
    def _count(j):
        idx = dstbuf[pl.ds(j * 16, 16)]
        plsc.addupdate_scatter(hist, [idx], ones16)

    pltpu.sync_copy(hist, hist_hbm.at[wid])


@functools.partial(
    pl.kernel,
    out_type=jax.ShapeDtypeStruct((NC, NPAD, DH), jnp.float32),
    mesh=_sc_mesh,
    scratch_types=[
        pltpu.VMEM((NB, K), jnp.int32),
        pltpu.VMEM((NB, K), jnp.int32),
        pltpu.VMEM((K, DH), jnp.float32),
        pltpu.VMEM((K, DH), jnp.float32),
        pltpu.VMEM((ZCH, DH), jnp.float32),
        pltpu.VMEM_SHARED((NPAD, DH), jnp.float32),
        pltpu.SemaphoreType.DMA,
        pltpu.SemaphoreType.DMA,
    ],
    compiler_params=pltpu.CompilerParams(use_tc_tiling_on_sc=False),
)
def _agg_sc(g_hbm, src_hbm, dst_hbm, zeros_hbm, part_hbm,
            srcbuf, dstbuf, gb0, gb1, zbuf, acc, sem0, sem1):
    cid = lax.axis_index("c")
    sid = lax.axis_index("s")

    pltpu.sync_copy(src_hbm.at[sid], srcbuf)
    pltpu.sync_copy(dst_hbm.at[sid], dstbuf)

    pltpu.sync_copy(zeros_hbm, zbuf)
    row0 = sid * ROWS_PER_TILE
    for kk in range(ROWS_PER_TILE // ZCH):
        pltpu.sync_copy(zbuf, acc.at[pl.ds(row0 + kk * ZCH, ZCH)])
    plsc.subcore_barrier()

    gplane = g_hbm.at[cid]

    pltpu.async_copy(gplane.at[srcbuf.at[0]], gb0, sem0)

    @pl.loop(0, NB // 2)
    def _pipe(t):
        j0 = 2 * t
        pltpu.make_async_copy(gplane.at[srcbuf.at[j0]], gb0, sem0).wait()
        pltpu.async_copy(gplane.at[srcbuf.at[j0 + 1]], gb1, sem1)
        pltpu.sync_copy(gb0, acc.at[dstbuf.at[j0]], add=True)
        pltpu.make_async_copy(gplane.at[srcbuf.at[j0 + 1]], gb1, sem1).wait()

        @pl.when(j0 + 2 < NB)
        def _prefetch():
            pltpu.async_copy(gplane.at[srcbuf.at[j0 + 2]], gb0, sem0)

        pltpu.sync_copy(gb1, acc.at[dstbuf.at[j0 + 1]], add=True)

    plsc.subcore_barrier()

    for kk in range(ROWS_PER_TILE // ZCH):
        r = row0 + kk * ZCH
        pltpu.sync_copy(acc.at[pl.ds(r, ZCH)], zbuf)
        pltpu.sync_copy(zbuf, part_hbm.at[cid, pl.ds(r, ZCH)])


_R = 1000


def _t1_body(degp_ref, x_ref, w_ref, dinv_ref, g_ref):
    deg = jnp.sum(degp_ref[...], axis=0) + 1.0
    dinv = lax.rsqrt(deg)
    h = jnp.dot(x_ref[...], w_ref[...], preferred_element_type=jnp.float32)
    g = dinv * h
    dinv_ref[...] = dinv
    g_ref[0] = g[:, :DH]
    g_ref[1] = g[:, DH:]


_t1 = pl.pallas_call(
    _t1_body,
    grid=(N_NODES // _R,),
    in_specs=[
        pl.BlockSpec((NW, _R, 1), lambda i: (0, i, 0)),
        pl.BlockSpec((_R, D), lambda i: (i, 0)),
        pl.BlockSpec((D, D), lambda i: (0, 0)),
    ],
    out_specs=[
        pl.BlockSpec((_R, 1), lambda i: (i, 0)),
        pl.BlockSpec((NC, _R, DH), lambda i: (0, i, 0)),
    ],
    out_shape=[
        jax.ShapeDtypeStruct((N_NODES, 1), jnp.float32),
        jax.ShapeDtypeStruct((NC, N_NODES, DH), jnp.float32),
    ],
)


def _t2_body(part_ref, g_ref, dinv_ref, b_ref, w_ref, out_ref):
    agg = part_ref[...] + g_ref[...]
    dinv = dinv_ref[...]
    z3 = jnp.maximum(dinv[None] * agg + b_ref[...], 0.0)
    z = jnp.concatenate([z3[0], z3[1]], axis=1)
    h = jnp.dot(z, w_ref[...], preferred_element_type=jnp.float32)
    g2 = dinv * h
    out_ref[0] = g2[:, :DH]
    out_ref[1] = g2[:, DH:]


_t2 = pl.pallas_call(
    _t2_body,
    grid=(N_NODES // _R,),
    in_specs=[
        pl.BlockSpec((NC, _R, DH), lambda i: (0, i, 0)),
        pl.BlockSpec((NC, _R, DH), lambda i: (0, i, 0)),
        pl.BlockSpec((_R, 1), lambda i: (i, 0)),
        pl.BlockSpec((NC, 1, DH), lambda i: (0, 0, 0)),
        pl.BlockSpec((D, D), lambda i: (0, 0)),
    ],
    out_specs=pl.BlockSpec((NC, _R, DH), lambda i: (0, i, 0)),
    out_shape=jax.ShapeDtypeStruct((NC, N_NODES, DH), jnp.float32),
)


def _t3_body(part_ref, g_ref, dinv_ref, b_ref, out_ref):
    agg = part_ref[...] + g_ref[...]
    z3 = jnp.maximum(dinv_ref[...][None] * agg + b_ref[...], 0.0)
    out_ref[...] = jnp.concatenate([z3[0], z3[1]], axis=1)


_t3 = pl.pallas_call(
    _t3_body,
    grid=(N_NODES // _R,),
    in_specs=[
        pl.BlockSpec((NC, _R, DH), lambda i: (0, i, 0)),
        pl.BlockSpec((NC, _R, DH), lambda i: (0, i, 0)),
        pl.BlockSpec((_R, 1), lambda i: (i, 0)),
        pl.BlockSpec((NC, 1, DH), lambda i: (0, 0, 0)),
    ],
    out_specs=pl.BlockSpec((_R, D), lambda i: (i, 0)),
    out_shape=jax.ShapeDtypeStruct((N_NODES, D), jnp.float32),
)


def kernel(x, edge_index, W1, b1, W2, b2):
    src = edge_index[0].astype(jnp.int32)
    dst = edge_index[1].astype(jnp.int32)
    src3 = src.reshape(NS, NB, K)
    dst3 = dst.reshape(NS, NB, K)
    dst2 = dst.reshape(NW, EPW)
    zeros = jnp.zeros((ZCH, DH), jnp.float32)

    hist = _deg_sc(dst2)
    degp = hist.reshape(NW, N_NODES, 1)
    dinv, g1 = _t1(degp, x, W1)
    p1 = _agg_sc(g1, src3, dst3, zeros)
    g2 = _t2(p1, g1, dinv, b1.reshape(NC, 1, DH), W2)
    p2 = _agg_sc(g2, src3, dst3, zeros)
    return _t3(p2, g2, dinv, b2.reshape(NC, 1, DH))
